# baseline (speedup 1.0000x reference)
import functools

import jax
import jax.numpy as jnp
import numpy as np
from jax import lax
from jax.experimental import pallas as pl
from jax.experimental.pallas import tpu as pltpu
from jax.experimental.pallas import tpu_sc as plsc

N = 10000
E = 160000
F = 128
NRBF = 64

N_PAD = 10240
E_PAD = 163840
NC, NS = 2, 16
NW = NC * NS
EW = E_PAD // NW
CH = 128
NCHUNK = EW // CH
CH_S = 80
NCHUNK_S = EW // CH_S
CH_F = 80
NCHUNK_F = EW // CH_F
RPT = N_PAD // NS

f32 = jnp.float32


@functools.cache
def _sc_mesh():
    return plsc.VectorSubcoreMesh(core_axis_name="c", subcore_axis_name="s",
                                  num_cores=NC, num_subcores=NS)


def _rows_binop(dst_ref, src_ref, op, n_rows, width=F):
    nv = width // 16

    def body(r, _):
        for cb in range(nv):
            sl = pl.ds(cb * 16, 16)
            dst_ref[r, sl] = op(dst_ref[r, sl], src_ref[r, sl])
        return 0

    lax.fori_loop(0, n_rows, body, 0, unroll=8)


def _fill(ref, value, n_rows, width=F):
    nv = width // 16
    val = jnp.full((16,), value, f32)

    def body(r, _):
        for cb in range(nv):
            ref[r, pl.ds(cb * 16, 16)] = val
        return 0

    lax.fori_loop(0, n_rows, body, 0, unroll=8)


def _zero_acc(acc, zbuf, rbase, width=F):
    _fill(zbuf, 0.0, CH, width)
    for k in range(RPT // CH):
        pltpu.sync_copy(zbuf, acc.at[pl.ds(rbase + k * CH, CH)])


def _copy_out_acc(acc, buf, out, core, rbase, width=F):
    for k in range(RPT // CH):
        r0 = rbase + k * CH
        pltpu.sync_copy(acc.at[pl.ds(r0, CH)], buf)
        pltpu.sync_copy(buf, out.at[core, pl.ds(r0, CH)])


def _worker():
    c = lax.axis_index("c")
    s = lax.axis_index("s")
    wid = c * NS + s
    return c, s, wid


@functools.cache
def _sc_gmul_segsum_k():
    return pl.kernel(
        _sc_gmul_segsum_body,
        out_type=jax.ShapeDtypeStruct((NC, N_PAD, F), f32),
        mesh=_sc_mesh(),
        scratch_types=[
            pltpu.VMEM_SHARED((N_PAD, F), f32),
            pltpu.VMEM((NCHUNK_F, CH_F), jnp.int32),
            pltpu.VMEM((NCHUNK_F, CH_F), jnp.int32),
            pltpu.VMEM((CH_F, F), f32),
            pltpu.VMEM((CH_F, F), f32),
            pltpu.VMEM((CH_F, F), f32),
            pltpu.SemaphoreType.DMA,
            pltpu.SemaphoreType.DMA,
        ],
    )


def _sc_gmul_segsum(*args):
    return _sc_gmul_segsum_k()(*args)


def _sc_gmul_segsum_body(tab, w, src3, dst3, out, acc, si2, di2,
                         r0, r1, w0, gs0, gs1):
    c, s, wid = _worker()
    ebase = wid * EW
    rbase = s * RPT
    rows = (r0, r1)
    gsem = (gs0, gs1)

    _fill(r0, 0.0, CH_F)
    for k in range(RPT // CH_F):
        pltpu.sync_copy(r0, acc.at[pl.ds(rbase + k * CH_F, CH_F)])
    pltpu.sync_copy(src3.at[wid], si2)
    pltpu.sync_copy(dst3.at[wid], di2)

    def g_start(j, b):
        pltpu.async_copy(tab.at[si2.at[j]], rows[b], gsem[b])

    def g_wait(j, b):
        pltpu.make_async_copy(tab.at[si2.at[j]], rows[b], gsem[b]).wait()

    g_start(0, 0)
    g_start(1, 1)
    plsc.subcore_barrier()

    def pair(g, _):
        for b in range(2):
            j = g * 2 + b
            pltpu.sync_copy(w.at[pl.ds(ebase + j * CH_F, CH_F)], w0)
            g_wait(j, b)
            _rows_binop(rows[b], w0, lax.mul, CH_F)
            pltpu.sync_copy(rows[b], acc.at[di2.at[j]], add=True)

            @pl.when(j + 2 < NCHUNK_F)
            def _(j=j, b=b):
                g_start(j + 2, b)

        return 0

    lax.fori_loop(0, NCHUNK_F // 2, pair, 0)
    plsc.subcore_barrier()
    for k in range(RPT // CH_F):
        rr = rbase + k * CH_F
        pltpu.sync_copy(acc.at[pl.ds(rr, CH_F)], r0)
        pltpu.sync_copy(r0, out.at[c, pl.ds(rr, CH_F)])


@functools.cache
def _sc_gather_mul_k():
    return pl.kernel(
        _sc_gather_mul_body,
        out_type=jax.ShapeDtypeStruct((E_PAD, F), f32),
        mesh=_sc_mesh(),
        scratch_types=[
            pltpu.VMEM((NCHUNK, CH), jnp.int32),
            pltpu.VMEM((CH, F), f32),
            pltpu.VMEM((CH, F), f32),
            pltpu.VMEM((CH, F), f32),
            pltpu.VMEM((CH, F), f32),
            pltpu.VMEM((CH, F), f32),
            pltpu.VMEM((CH, F), f32),
            pltpu.SemaphoreType.DMA,
            pltpu.SemaphoreType.DMA,
            pltpu.SemaphoreType.DMA,
            pltpu.SemaphoreType.DMA,
            pltpu.SemaphoreType.DMA,
            pltpu.SemaphoreType.DMA,
        ],
    )


def _sc_gather_mul(*args):
    return _sc_gather_mul_k()(*args)


def _mul_to(out_ref, a_ref, b_ref, n_rows, width=F):
    nv = width // 16

    def body(r, _):
        for cb in range(nv):
            sl = pl.ds(cb * 16, 16)
            out_ref[r, sl] = a_ref[r, sl] * b_ref[r, sl]
        return 0

    lax.fori_loop(0, n_rows, body, 0, unroll=8)


def _sc_gather_mul_body(tab, w, src3, out, si2, r0, r1, w0, w1, o0, o1,
                        gs0, gs1, ws0, ws1, os0, os1):
    c, s, wid = _worker()
    ebase = wid * EW
    rows = (r0, r1)
    wb = (w0, w1)
    ob = (o0, o1)
    gsem = (gs0, gs1)
    wsem = (ws0, ws1)
    osem = (os0, os1)

    pltpu.sync_copy(src3.at[wid], si2)

    def g_start(j, b):
        pltpu.async_copy(tab.at[si2.at[j]], rows[b], gsem[b])
        pltpu.async_copy(w.at[pl.ds(ebase + j * CH, CH)], wb[b], wsem[b])

    def g_wait(j, b):
        pltpu.make_async_copy(tab.at[si2.at[j]], rows[b], gsem[b]).wait()
        pltpu.make_async_copy(w.at[pl.ds(ebase + j * CH, CH)], wb[b],
                              wsem[b]).wait()

    g_start(0, 0)
    g_start(1, 1)

    def pair(g, _):
        for b in range(2):
            j = g * 2 + b

            @pl.when(j >= 2)
            def _(j=j, b=b):
                pltpu.make_async_copy(
                    ob[b], out.at[pl.ds(ebase + (j - 2) * CH, CH)],
                    osem[b]).wait()

            g_wait(j, b)
            _mul_to(ob[b], rows[b], wb[b], CH)

            @pl.when(j + 2 < NCHUNK)
            def _(j=j, b=b):
                g_start(j + 2, b)

            pltpu.async_copy(ob[b], out.at[pl.ds(ebase + j * CH, CH)], osem[b])
        return 0

    lax.fori_loop(0, NCHUNK // 2, pair, 0)
    for b in range(2):
        j = NCHUNK - 2 + b
        pltpu.make_async_copy(ob[b], out.at[pl.ds(ebase + j * CH, CH)],
                              osem[b]).wait()


@functools.cache
def _sc_segsum_k():
    return pl.kernel(
        _sc_segsum_body,
        out_type=jax.ShapeDtypeStruct((NC, N_PAD, F), f32),
        mesh=_sc_mesh(),
        scratch_types=[
            pltpu.VMEM_SHARED((N_PAD, F), f32),
            pltpu.VMEM((NCHUNK_S, CH_S), jnp.int32),
            pltpu.VMEM((CH_S, F), f32),
            pltpu.VMEM((CH_S, F), f32),
            pltpu.SemaphoreType.DMA,
            pltpu.SemaphoreType.DMA,
        ],
    )


def _sc_segsum(*args):
    return _sc_segsum_k()(*args)


def _sc_segsum_body(m, dst3, out, acc, di2, rows0, rows1, gs0, gs1):
    c, s, wid = _worker()
    ebase = wid * EW
    rbase = s * RPT
    rows = (rows0, rows1)
    gsem = (gs0, gs1)

    _fill(rows0, 0.0, CH_S)
    for k in range(RPT // CH_S):
        pltpu.sync_copy(rows0, acc.at[pl.ds(rbase + k * CH_S, CH_S)])
    pltpu.sync_copy(dst3.at[wid], di2)

    def g_start(j, b):
        pltpu.async_copy(m.at[pl.ds(ebase + j * CH_S, CH_S)], rows[b], gsem[b])

    def g_wait(j, b):
        pltpu.make_async_copy(m.at[pl.ds(ebase + j * CH_S, CH_S)], rows[b],
                              gsem[b]).wait()

    g_start(0, 0)
    g_start(1, 1)
    plsc.subcore_barrier()

    def pair(g, _):
        for b in range(2):
            j = g * 2 + b
            g_wait(j, b)
            pltpu.sync_copy(rows[b], acc.at[di2.at[j]], add=True)

            @pl.when(j + 2 < NCHUNK_S)
            def _(j=j, b=b):
                g_start(j + 2, b)

        return 0

    lax.fori_loop(0, NCHUNK_S // 2, pair, 0)
    plsc.subcore_barrier()
    for k in range(RPT // CH_S):
        r0 = rbase + k * CH_S
        pltpu.sync_copy(acc.at[pl.ds(r0, CH_S)], rows0)
        pltpu.sync_copy(rows0, out.at[c, pl.ds(r0, CH_S)])


def _make_sc_gather2_impl(sub=False):
    scratch = [
        pltpu.VMEM((NCHUNK, CH), jnp.int32),
        pltpu.VMEM((NCHUNK, CH), jnp.int32),
        pltpu.VMEM((CH, F), f32),
        pltpu.VMEM((CH, F), f32),
        pltpu.VMEM((CH, F), f32),
        pltpu.VMEM((CH, F), f32),
        pltpu.VMEM((CH, F), f32),
        pltpu.VMEM((CH, F), f32),
        pltpu.SemaphoreType.DMA,
        pltpu.SemaphoreType.DMA,
        pltpu.SemaphoreType.DMA,
        pltpu.SemaphoreType.DMA,
        pltpu.SemaphoreType.DMA,
        pltpu.SemaphoreType.DMA,
    ]

    def body(ta, tb, src3, dst3, out, si2, di2, a0, a1, b0, b1, o0, o1,
             as0, as1, bs0, bs1, os0, os1):
        c, s, wid = _worker()
        ebase = wid * EW
        ab = (a0, a1)
        bb = (b0, b1)
        ob = (o0, o1)
        asem = (as0, as1)
        bsem = (bs0, bs1)
        osem = (os0, os1)

        pltpu.sync_copy(src3.at[wid], si2)
        pltpu.sync_copy(dst3.at[wid], di2)

        def g_start(j, b):
            pltpu.async_copy(ta.at[si2.at[j]], ab[b], asem[b])
            pltpu.async_copy(tb.at[di2.at[j]], bb[b], bsem[b])

        def g_wait(j, b):
            pltpu.make_async_copy(ta.at[si2.at[j]], ab[b], asem[b]).wait()
            pltpu.make_async_copy(tb.at[di2.at[j]], bb[b], bsem[b]).wait()

        def op_to(out_ref, x_ref, y_ref):
            nv = F // 16

            def rbody(r, _):
                for cb in range(nv):
                    sl = pl.ds(cb * 16, 16)
                    if sub:
                        out_ref[r, sl] = x_ref[r, sl] - y_ref[r, sl]
                    else:
                        out_ref[r, sl] = x_ref[r, sl] + y_ref[r, sl]
                return 0

            lax.fori_loop(0, CH, rbody, 0, unroll=8)

        g_start(0, 0)
        g_start(1, 1)

        def pair(g, _):
            for b in range(2):
                j = g * 2 + b

                @pl.when(j >= 2)
                def _(j=j, b=b):
                    pltpu.make_async_copy(
                        ob[b], out.at[pl.ds(ebase + (j - 2) * CH, CH)],
                        osem[b]).wait()

                g_wait(j, b)
                op_to(ob[b], ab[b], bb[b])

                @pl.when(j + 2 < NCHUNK)
                def _(j=j, b=b):
                    g_start(j + 2, b)

                pltpu.async_copy(ob[b], out.at[pl.ds(ebase + j * CH, CH)],
                                 osem[b])
            return 0

        lax.fori_loop(0, NCHUNK // 2, pair, 0)
        for b in range(2):
            j = NCHUNK - 2 + b
            pltpu.make_async_copy(ob[b], out.at[pl.ds(ebase + j * CH, CH)],
                                  osem[b]).wait()

    return pl.kernel(
        body,
        out_type=jax.ShapeDtypeStruct((E_PAD, F), f32),
        mesh=_sc_mesh(),
        scratch_types=scratch,
    )


_make_sc_gather2 = functools.cache(_make_sc_gather2_impl)


def _sc_gather2(*args):
    return _make_sc_gather2(False)(*args)


def _sc_gather_diff(*args):
    return _make_sc_gather2(True)(*args)


@functools.cache
def _sc_ones_segsum_k():
    return pl.kernel(
        _sc_ones_segsum_body,
        out_type=jax.ShapeDtypeStruct((NC, N_PAD, F), f32),
        mesh=_sc_mesh(),
        scratch_types=[
            pltpu.VMEM_SHARED((N_PAD, F), f32),
            pltpu.VMEM((NCHUNK_S, CH_S), jnp.int32),
            pltpu.VMEM((CH_S, F), f32),
        ],
    )


def _sc_ones_segsum(*args):
    return _sc_ones_segsum_k()(*args)


def _sc_ones_segsum_body(dst3, out, acc, di2, rows_v):
    c, s, wid = _worker()
    rbase = s * RPT

    _fill(rows_v, 0.0, CH_S)
    for k in range(RPT // CH_S):
        pltpu.sync_copy(rows_v, acc.at[pl.ds(rbase + k * CH_S, CH_S)])
    pltpu.sync_copy(dst3.at[wid], di2)
    plsc.subcore_barrier()
    _fill(rows_v, 1.0, CH_S)

    def chunk(j, _):
        pltpu.sync_copy(rows_v, acc.at[di2.at[j]], add=True)
        return 0

    lax.fori_loop(0, NCHUNK_S, chunk, 0)
    plsc.subcore_barrier()
    for k in range(RPT // CH_S):
        r0 = rbase + k * CH_S
        pltpu.sync_copy(acc.at[pl.ds(r0, CH_S)], rows_v)
        pltpu.sync_copy(rows_v, out.at[c, pl.ds(r0, CH_S)])


B_N = 1024
B_E = 1024
GN = N_PAD // B_N
GE = E_PAD // B_E


def _row_spec(b, w):
    return pl.BlockSpec((b, w), lambda i: (i, 0))


def _full_spec(shape):
    nd = len(shape)
    return pl.BlockSpec(shape, lambda i: (0,) * nd)


def _tc_call(body, full_shapes, out_widths, grid, b_rows, in_widths,
             interpret=False):
    in_specs = [_row_spec(b_rows, w) for w in in_widths]
    in_specs += [_full_spec(s) for s in full_shapes]
    out_specs = [_row_spec(b_rows, w) for w in out_widths]
    return pl.pallas_call(
        body,
        grid=(grid,),
        in_specs=in_specs,
        out_specs=out_specs[0] if len(out_widths) == 1 else tuple(out_specs),
        out_shape=(
            jax.ShapeDtypeStruct((grid * b_rows, out_widths[0]), f32)
            if len(out_widths) == 1
            else tuple(jax.ShapeDtypeStruct((grid * b_rows, w), f32)
                       for w in out_widths)
        ),
        interpret=interpret,
    )


def _dot(a, b):
    return jnp.dot(a, b, preferred_element_type=f32)


def _sigmoid(v):
    return jax.nn.sigmoid(v)


def _silu(v):
    return v * jax.nn.sigmoid(v)


def _ssp_tc(v):
    return (jnp.log(1.0 + jnp.exp(-jnp.abs(v))) + jnp.maximum(v, 0.0)
            - np.float32(np.log(2.0)))


def _t1_body(x_ref, wnp, bnp, wm, bm, h_ref, lin_ref):
    h = _dot(x_ref[...], wnp[...]) + bnp[...]
    h_ref[...] = h
    lin_ref[...] = _dot(h, wm[...]) + bm[...]


def _prep_node(x, wnp, bnp, wm, bm, interpret=False):
    fn = _tc_call(_t1_body, [wnp.shape, bnp.shape, wm.shape, bm.shape],
                  [F, F], GN, B_N, [x.shape[1]], interpret)
    return fn(x, wnp, bnp, wm, bm)


def _t2_body(attr_ref, diff_ref,
             wg1, bg1, wg2, bg2,
             wpf0, bpf0, wrf0, wf20, bf20,
             wpf1, bpf1, wrf1, wf21, bf21,
             eg_ref, f0_ref, f1_ref, d_ref):
    a = attr_ref[...]
    diff = diff_ref[...]
    d2 = jnp.sum(diff * diff, axis=1, keepdims=True)
    d = jnp.sqrt(d2 + 1e-12)
    centers = (4.0 / 63.0) * lax.broadcasted_iota(jnp.int32, (1, NRBF), 1).astype(f32)
    rbf = jnp.exp(-10.0 * (d - centers) ** 2)

    eg1 = jax.nn.relu(_dot(a, wg1[...]) + bg1[...])
    eg_ref[...] = _dot(eg1, wg2[...]) + bg2[...]

    ef0 = _dot(a, wpf0[...]) + _dot(rbf, wrf0[...]) + bpf0[...]
    f0_ref[...] = _dot(_ssp_tc(ef0), wf20[...]) + bf20[...]
    ef1 = _dot(a, wpf1[...]) + _dot(rbf, wrf1[...]) + bpf1[...]
    f1_ref[...] = _dot(_ssp_tc(ef1), wf21[...]) + bf21[...]

    d_ref[...] = jnp.concatenate(
        [d, d2, jnp.zeros((d.shape[0], 14), f32)], axis=1)


def _prep_edge(attr, diff, weights, interpret=False):
    full_shapes = [w.shape for w in weights]
    fn = _tc_call(_t2_body, full_shapes, [F, F, F, 16], GE, B_E,
                  [attr.shape[1], F], interpret)
    return fn(attr, diff, *weights)


def _t3_body(h_ref, m0_ref, m1_ref, dg0_ref, dg1_ref,
             wgi, bgi, wgh, bgh, wnx, bnx, h_out, lin_out):
    h = h_ref[...]
    deg = jnp.maximum(dg0_ref[...][:, :1] + dg1_ref[...][:, :1], 1.0)
    m2d = (m0_ref[...] + m1_ref[...]) / deg
    gi = _dot(m2d, wgi[...]) + bgi[...]
    gh = _dot(h, wgh[...]) + bgh[...]
    r = _sigmoid(gi[:, :F] + gh[:, :F])
    z = _sigmoid(gi[:, F:2 * F] + gh[:, F:2 * F])
    a = jnp.tanh(gi[:, 2 * F:] + r * gh[:, 2 * F:])
    hn = (1.0 - z) * a + z * h
    h_out[...] = hn
    lin_out[...] = _dot(hn, wnx[...]) + bnx[...]


def _gru_step(h, m0, m1, dg0, dg1, wgi, bgi, wgh, bgh, wnx, bnx, interpret=False):
    fn = _tc_call(_t3_body,
                  [wgi.shape, bgi.shape, wgh.shape, bgh.shape, wnx.shape, bnx.shape],
                  [F, F], GN, B_N, [F, F, F, 16, 16], interpret)
    return fn(h, m0, m1, dg0, dg1, wgi, bgi, wgh, bgh, wnx, bnx)


def _t4_body(xs_ref, a0_ref, a1_ref, wu1, bu1, wu2, bu2, wnx, bnx, xs_out, lin_out):
    agg = a0_ref[...] + a1_ref[...]
    t = _ssp_tc(_dot(agg, wu1[...]) + bu1[...])
    xs = xs_ref[...] + _dot(t, wu2[...]) + bu2[...]
    xs_out[...] = xs
    lin_out[...] = _dot(xs, wnx[...]) + bnx[...]


def _schnet_node(xs, a0, a1, wu1, bu1, wu2, bu2, wnx, bnx, interpret=False):
    fn = _tc_call(_t4_body,
                  [wu1.shape, bu1.shape, wu2.shape, bu2.shape, wnx.shape, bnx.shape],
                  [F, F], GN, B_N, [F, F, F], interpret)
    return fn(xs, a0, a1, wu1, bu1, wu2, bu2, wnx, bnx)


def _t5_body(xe_ref, a0_ref, a1_ref, wh1a, wh1b, bh1, wh2, bh2, xe_out):
    xe = xe_ref[...]
    agg = a0_ref[...] + a1_ref[...]
    u = _silu(_dot(xe, wh1a[...]) + _dot(agg, wh1b[...]) + bh1[...])
    xe_out[...] = xe + _dot(u, wh2[...]) + bh2[...]


def _egnn_node(xe, a0, a1, wh1a, wh1b, bh1, wh2, bh2, interpret=False):
    fn = _tc_call(_t5_body,
                  [wh1a.shape, wh1b.shape, bh1.shape, wh2.shape, bh2.shape],
                  [F], GN, B_N, [F, F, F], interpret)
    return fn(xe, a0, a1, wh1a, wh1b, bh1, wh2, bh2)


def _tnm2_body(x_ref, wa, wb, a_out, b_out):
    x = x_ref[...]
    a_out[...] = _dot(x, wa[...])
    b_out[...] = _dot(x, wb[...])


def _node_ab(x, wa, wb, interpret=False):
    fn = _tc_call(_tnm2_body, [wa.shape, wb.shape], [F, F], GN, B_N, [F],
                  interpret)
    return fn(x, wa, wb)


def _t6_body(g_ref, attr_ref, d_ref, wpe, bpe, wre, wd2r, we2, be2, m_ref):
    d = d_ref[...][:, :1]
    d2 = d_ref[...][:, 1:2]
    centers = (4.0 / 63.0) * lax.broadcasted_iota(jnp.int32, (1, NRBF), 1).astype(f32)
    rbf = jnp.exp(-10.0 * (d - centers) ** 2)
    a = attr_ref[...]
    efe = _dot(a, wpe[...]) + _dot(rbf, wre[...]) + bpe[...] + d2 * wd2r[...]
    u = _silu(g_ref[...] + efe)
    m_ref[...] = _silu(_dot(u, we2[...]) + be2[...])


def _egnn_msg(g, attr, d16, wpe, bpe, wre, wd2r, we2, be2, interpret=False):
    fn = _tc_call(_t6_body,
                  [wpe.shape, bpe.shape, wre.shape, wd2r.shape, we2.shape,
                   be2.shape],
                  [F], GE, B_E, [F, attr.shape[1], 16], interpret)
    return fn(g, attr, d16, wpe, bpe, wre, wd2r, we2, be2)


def _t7_body(h_ref, xs_ref, xe_ref, wg3a, wg3b, bg3, wg23a, wg23b, bg23,
             wda, wdb, bd1, p_out, q_out):
    h = h_ref[...]
    xs = xs_ref[...]
    xe = xe_ref[...]
    g3 = _sigmoid(_dot(xs, wg3a[...]) + _dot(xe, wg3b[...]) + bg3[...])
    x3d = g3 * xs + (1.0 - g3) * xe
    g23 = _sigmoid(_dot(h, wg23a[...]) + _dot(x3d, wg23b[...]) + bg23[...])
    x3d = g23 * x3d + (1.0 - g23) * h
    p_out[...] = _dot(x3d, wda[...])
    q_out[...] = _dot(x3d, wdb[...]) + bd1[...]


def _fuse_gates(h, xs, xe, wg3a, wg3b, bg3, wg23a, wg23b, bg23, wda, wdb, bd1,
                interpret=False):
    fn = _tc_call(_t7_body,
                  [wg3a.shape, wg3b.shape, bg3.shape, wg23a.shape, wg23b.shape,
                   bg23.shape, wda.shape, wdb.shape, bd1.shape],
                  [F, F], GN, B_N, [F, F, F], interpret)
    return fn(h, xs, xe, wg3a, wg3b, bg3, wg23a, wg23b, bg23, wda, wdb, bd1)


def _t8_body(pair_ref, d_ref, wd2, bd2, out_ref):
    i = pl.program_id(0)
    v = jax.nn.relu(pair_ref[...])
    pred = jnp.sum(v * wd2[...], axis=1, keepdims=True) + bd2[...]
    err = pred - d_ref[...][:, :1]
    gidx = i * B_E + lax.broadcasted_iota(jnp.int32, (B_E, 1), 0)
    msk = (gidx < E).astype(f32)
    part = jnp.sum(err * err * msk)

    @pl.when(i == 0)
    def _():
        out_ref[0, 0] = 0.0

    out_ref[0, 0] += part


def _loss_sum(pair, d16, wd2, bd2, interpret=False):
    fn = pl.pallas_call(
        _t8_body,
        grid=(GE,),
        in_specs=[
            _row_spec(B_E, F),
            _row_spec(B_E, 16),
            _full_spec(wd2.shape),
            _full_spec(bd2.shape),
        ],
        out_specs=pl.BlockSpec((1, 1), lambda i: (0, 0), memory_space=pltpu.SMEM),
        out_shape=jax.ShapeDtypeStruct((1, 1), f32),
        interpret=interpret,
    )
    return fn(pair, d16, wd2, bd2)


def kernel(x, edge_index, edge_attr, pos, params):
    p = params
    r1 = lambda b: b.reshape(1, -1)

    xp = jnp.pad(x, ((0, N_PAD - N), (0, 0)))
    attr_p = jnp.pad(edge_attr, ((0, E_PAD - E), (0, 0)))
    src_p = jnp.pad(edge_index[0].astype(jnp.int32), (0, E_PAD - E),
                    constant_values=N_PAD - 1)
    dst_p = jnp.pad(edge_index[1].astype(jnp.int32), (0, E_PAD - E),
                    constant_values=N_PAD - 1)
    pos128 = jnp.pad(pos, ((0, N_PAD - N), (0, F - 3)))
    perm = jnp.argsort(src_p)
    src_p = src_p[perm]
    dst_p = dst_p[perm]
    attr_p = attr_p[perm]
    src3 = src_p.reshape(NW, NCHUNK, CH)
    dst3 = dst_p.reshape(NW, NCHUNK, CH)
    dst3s = dst_p.reshape(NW, NCHUNK_S, CH_S)
    src3f = src_p.reshape(NW, NCHUNK_F, CH_F)
    dst3f = dst_p.reshape(NW, NCHUNK_F, CH_F)

    Wp, bp = p["edge_proj_3d"]["w"], p["edge_proj_3d"]["b"]
    t2_w = [p["edge_gate1"]["w"], r1(p["edge_gate1"]["b"]),
            p["edge_gate2"]["w"], r1(p["edge_gate2"]["b"])]
    for l in p["schnet"]:
        W1, b1 = l["filt1"]["w"], l["filt1"]["b"]
        t2_w += [Wp @ W1[:F], r1(bp @ W1[:F] + b1), W1[F:],
                 l["filt2"]["w"], r1(l["filt2"]["b"])]
    egnn_w = []
    for l in p["egnn"]:
        W1, b1 = l["e1"]["w"], l["e1"]["b"]
        Wtop, Wbot = W1[2 * F + 1:3 * F + 1], W1[3 * F + 1:]
        egnn_w.append([Wp @ Wtop, r1(bp @ Wtop + b1), Wbot, W1[2 * F:2 * F + 1]])

    h, lin = _prep_node(xp, p["node_proj"]["w"], r1(p["node_proj"]["b"]),
                        p["msg"]["w"], r1(p["msg"]["b"]))

    pdiff = _sc_gather_diff(pos128, pos128, src3, dst3)
    degf = _sc_ones_segsum(dst3s)
    dg0, dg1 = degf[0, :, :16], degf[1, :, :16]

    eg, filt0, filt1, d16 = _prep_edge(attr_p, pdiff, t2_w)
    filts = [filt0, filt1]

    winp0, binp0 = p["schnet"][0]["inp"]["w"], r1(p["schnet"][0]["inp"]["b"])
    for step in range(3):
        mh = _sc_gmul_segsum(lin, eg, src3f, dst3f)
        if step < 2:
            wnx, bnx = p["msg"]["w"], r1(p["msg"]["b"])
        else:
            wnx, bnx = winp0, binp0
        h, lin = _gru_step(h, mh[0], mh[1], dg0, dg1,
                           p["gru_i"]["w"], r1(p["gru_i"]["b"]),
                           p["gru_h"]["w"], r1(p["gru_h"]["b"]), wnx, bnx)

    xs = h
    winp1, binp1 = p["schnet"][1]["inp"]["w"], r1(p["schnet"][1]["inp"]["b"])
    for li, l in enumerate(p["schnet"]):
        ah = _sc_gmul_segsum(lin, filts[li], src3f, dst3f)
        xs, lin = _schnet_node(xs, ah[0], ah[1],
                               l["upd1"]["w"], r1(l["upd1"]["b"]),
                               l["upd2"]["w"], r1(l["upd2"]["b"]),
                               winp1, binp1)

    xe = h
    W1_0 = p["egnn"][0]["e1"]["w"]
    A, B = _node_ab(h, W1_0[:F], W1_0[F:2 * F])
    for li, l in enumerate(p["egnn"]):
        G = _sc_gather2(A, B, src3, dst3)
        mij = _egnn_msg(G, attr_p, d16, *egnn_w[li],
                        l["e2"]["w"], r1(l["e2"]["b"]))
        agg = _sc_segsum(mij, dst3s)
        Wh1 = l["h1"]["w"]
        xe = _egnn_node(xe, agg[0], agg[1], Wh1[:F], Wh1[F:], r1(l["h1"]["b"]),
                        l["h2"]["w"], r1(l["h2"]["b"]))
        if li == 0:
            W1_1 = p["egnn"][1]["e1"]["w"]
            A, B = _node_ab(xe, W1_1[:F], W1_1[F:2 * F])

    Wg3, Wg23, Wd1 = p["gate3"]["w"], p["gate23"]["w"], p["dist1"]["w"]
    P, Q = _fuse_gates(h, xs, xe, Wg3[:F], Wg3[F:], r1(p["gate3"]["b"]),
                       Wg23[:F], Wg23[F:], r1(p["gate23"]["b"]),
                       Wd1[:F], Wd1[F:], r1(p["dist1"]["b"]))
    pair = _sc_gather2(P, Q, src3, dst3)
    ssum = _loss_sum(pair, d16, r1(p["dist2"]["w"][:, 0]),
                     p["dist2"]["b"].reshape(1, 1))
    return ssum[0, 0] / np.float32(E)


# speedup vs baseline: 1.1209x; 1.1209x over previous
import functools

import jax
import jax.numpy as jnp
import numpy as np
from jax import lax
from jax.experimental import pallas as pl
from jax.experimental.pallas import tpu as pltpu
from jax.experimental.pallas import tpu_sc as plsc

N = 10000
E = 160000
F = 128
NRBF = 64

N_PAD = 10240
E_PAD = 163840
NC, NS = 2, 16
NW = NC * NS
EW = E_PAD // NW
CH = 128
NCHUNK = EW // CH
CH_S = 80
NCHUNK_S = EW // CH_S
CH_F = 64
NCHUNK_F = EW // CH_F
RPT = N_PAD // NS

f32 = jnp.float32


@functools.cache
def _sc_mesh():
    return plsc.VectorSubcoreMesh(core_axis_name="c", subcore_axis_name="s",
                                  num_cores=NC, num_subcores=NS)


def _rows_binop(dst_ref, src_ref, op, n_rows, width=F):
    nv = width // 16

    def body(r, _):
        for cb in range(nv):
            sl = pl.ds(cb * 16, 16)
            dst_ref[r, sl] = op(dst_ref[r, sl], src_ref[r, sl])
        return 0

    lax.fori_loop(0, n_rows, body, 0, unroll=8)


def _fill(ref, value, n_rows, width=F):
    nv = width // 16
    val = jnp.full((16,), value, f32)

    def body(r, _):
        for cb in range(nv):
            ref[r, pl.ds(cb * 16, 16)] = val
        return 0

    lax.fori_loop(0, n_rows, body, 0, unroll=8)


def _zero_acc(acc, zbuf, rbase, width=F):
    _fill(zbuf, 0.0, CH, width)
    for k in range(RPT // CH):
        pltpu.sync_copy(zbuf, acc.at[pl.ds(rbase + k * CH, CH)])


def _copy_out_acc(acc, buf, out, core, rbase, width=F):
    for k in range(RPT // CH):
        r0 = rbase + k * CH
        pltpu.sync_copy(acc.at[pl.ds(r0, CH)], buf)
        pltpu.sync_copy(buf, out.at[core, pl.ds(r0, CH)])


def _worker():
    c = lax.axis_index("c")
    s = lax.axis_index("s")
    wid = c * NS + s
    return c, s, wid


@functools.cache
def _sc_gmul_segsum_k():
    return pl.kernel(
        _sc_gmul_segsum_body,
        out_type=jax.ShapeDtypeStruct((NC, N_PAD, F), f32),
        mesh=_sc_mesh(),
        scratch_types=[
            pltpu.VMEM_SHARED((N_PAD, F), f32),
            pltpu.VMEM((NCHUNK_F, 2 * CH_F), jnp.int32),
            pltpu.VMEM((CH_F, F), f32),
            pltpu.VMEM((CH_F, F), f32),
            pltpu.VMEM((CH_F, F), f32),
            pltpu.VMEM((CH_F, F), f32),
            pltpu.SemaphoreType.DMA,
            pltpu.SemaphoreType.DMA,
            pltpu.SemaphoreType.DMA,
            pltpu.SemaphoreType.DMA,
        ],
    )


def _sc_gmul_segsum(*args):
    return _sc_gmul_segsum_k()(*args)


def _sc_gmul_segsum_body(tab, w, sd3, out, acc, sd2,
                         r0, r1, w0, w1, gs0, gs1, ws0, ws1):
    c, s, wid = _worker()
    ebase = wid * EW
    rbase = s * RPT
    rows = (r0, r1)
    wb = (w0, w1)
    gsem = (gs0, gs1)
    wsem = (ws0, ws1)

    _fill(r0, 0.0, CH_F)
    for k in range(RPT // CH_F):
        pltpu.sync_copy(r0, acc.at[pl.ds(rbase + k * CH_F, CH_F)])
    pltpu.sync_copy(sd3.at[wid], sd2)

    def g_start(j, b):
        pltpu.async_copy(tab.at[sd2.at[j, pl.ds(0, CH_F)]], rows[b], gsem[b])
        pltpu.async_copy(w.at[pl.ds(ebase + j * CH_F, CH_F)], wb[b], wsem[b])

    def g_wait(j, b):
        pltpu.make_async_copy(tab.at[sd2.at[j, pl.ds(0, CH_F)]], rows[b],
                              gsem[b]).wait()
        pltpu.make_async_copy(w.at[pl.ds(ebase + j * CH_F, CH_F)], wb[b],
                              wsem[b]).wait()

    g_start(0, 0)
    g_start(1, 1)
    plsc.subcore_barrier()

    def pair(g, _):
        for b in range(2):
            j = g * 2 + b
            g_wait(j, b)
            _rows_binop(rows[b], wb[b], lax.mul, CH_F)
            pltpu.sync_copy(rows[b], acc.at[sd2.at[j, pl.ds(CH_F, CH_F)]],
                            add=True)

            @pl.when(j + 2 < NCHUNK_F)
            def _(j=j, b=b):
                g_start(j + 2, b)

        return 0

    lax.fori_loop(0, NCHUNK_F // 2, pair, 0)
    plsc.subcore_barrier()
    for k in range(RPT // CH_F):
        rr = rbase + k * CH_F
        pltpu.sync_copy(acc.at[pl.ds(rr, CH_F)], r0)
        pltpu.sync_copy(r0, out.at[c, pl.ds(rr, CH_F)])


@functools.cache
def _sc_gather_mul_k():
    return pl.kernel(
        _sc_gather_mul_body,
        out_type=jax.ShapeDtypeStruct((E_PAD, F), f32),
        mesh=_sc_mesh(),
        scratch_types=[
            pltpu.VMEM((NCHUNK, CH), jnp.int32),
            pltpu.VMEM((CH, F), f32),
            pltpu.VMEM((CH, F), f32),
            pltpu.VMEM((CH, F), f32),
            pltpu.VMEM((CH, F), f32),
            pltpu.VMEM((CH, F), f32),
            pltpu.VMEM((CH, F), f32),
            pltpu.SemaphoreType.DMA,
            pltpu.SemaphoreType.DMA,
            pltpu.SemaphoreType.DMA,
            pltpu.SemaphoreType.DMA,
            pltpu.SemaphoreType.DMA,
            pltpu.SemaphoreType.DMA,
        ],
    )


def _sc_gather_mul(*args):
    return _sc_gather_mul_k()(*args)


def _mul_to(out_ref, a_ref, b_ref, n_rows, width=F):
    nv = width // 16

    def body(r, _):
        for cb in range(nv):
            sl = pl.ds(cb * 16, 16)
            out_ref[r, sl] = a_ref[r, sl] * b_ref[r, sl]
        return 0

    lax.fori_loop(0, n_rows, body, 0, unroll=8)


def _sc_gather_mul_body(tab, w, src3, out, si2, r0, r1, w0, w1, o0, o1,
                        gs0, gs1, ws0, ws1, os0, os1):
    c, s, wid = _worker()
    ebase = wid * EW
    rows = (r0, r1)
    wb = (w0, w1)
    ob = (o0, o1)
    gsem = (gs0, gs1)
    wsem = (ws0, ws1)
    osem = (os0, os1)

    pltpu.sync_copy(src3.at[wid], si2)

    def g_start(j, b):
        pltpu.async_copy(tab.at[si2.at[j]], rows[b], gsem[b])
        pltpu.async_copy(w.at[pl.ds(ebase + j * CH, CH)], wb[b], wsem[b])

    def g_wait(j, b):
        pltpu.make_async_copy(tab.at[si2.at[j]], rows[b], gsem[b]).wait()
        pltpu.make_async_copy(w.at[pl.ds(ebase + j * CH, CH)], wb[b],
                              wsem[b]).wait()

    g_start(0, 0)
    g_start(1, 1)

    def pair(g, _):
        for b in range(2):
            j = g * 2 + b

            @pl.when(j >= 2)
            def _(j=j, b=b):
                pltpu.make_async_copy(
                    ob[b], out.at[pl.ds(ebase + (j - 2) * CH, CH)],
                    osem[b]).wait()

            g_wait(j, b)
            _mul_to(ob[b], rows[b], wb[b], CH)

            @pl.when(j + 2 < NCHUNK)
            def _(j=j, b=b):
                g_start(j + 2, b)

            pltpu.async_copy(ob[b], out.at[pl.ds(ebase + j * CH, CH)], osem[b])
        return 0

    lax.fori_loop(0, NCHUNK // 2, pair, 0)
    for b in range(2):
        j = NCHUNK - 2 + b
        pltpu.make_async_copy(ob[b], out.at[pl.ds(ebase + j * CH, CH)],
                              osem[b]).wait()


@functools.cache
def _sc_segsum_k():
    return pl.kernel(
        _sc_segsum_body,
        out_type=jax.ShapeDtypeStruct((NC, N_PAD, F), f32),
        mesh=_sc_mesh(),
        scratch_types=[
            pltpu.VMEM_SHARED((N_PAD, F), f32),
            pltpu.VMEM((NCHUNK_S, CH_S), jnp.int32),
            pltpu.VMEM((CH_S, F), f32),
            pltpu.VMEM((CH_S, F), f32),
            pltpu.SemaphoreType.DMA,
            pltpu.SemaphoreType.DMA,
        ],
    )


def _sc_segsum(*args):
    return _sc_segsum_k()(*args)


def _sc_segsum_body(m, dst3, out, acc, di2, rows0, rows1, gs0, gs1):
    c, s, wid = _worker()
    ebase = wid * EW
    rbase = s * RPT
    rows = (rows0, rows1)
    gsem = (gs0, gs1)

    _fill(rows0, 0.0, CH_S)
    for k in range(RPT // CH_S):
        pltpu.sync_copy(rows0, acc.at[pl.ds(rbase + k * CH_S, CH_S)])
    pltpu.sync_copy(dst3.at[wid], di2)

    def g_start(j, b):
        pltpu.async_copy(m.at[pl.ds(ebase + j * CH_S, CH_S)], rows[b], gsem[b])

    def g_wait(j, b):
        pltpu.make_async_copy(m.at[pl.ds(ebase + j * CH_S, CH_S)], rows[b],
                              gsem[b]).wait()

    g_start(0, 0)
    g_start(1, 1)
    plsc.subcore_barrier()

    def pair(g, _):
        for b in range(2):
            j = g * 2 + b
            g_wait(j, b)
            pltpu.sync_copy(rows[b], acc.at[di2.at[j]], add=True)

            @pl.when(j + 2 < NCHUNK_S)
            def _(j=j, b=b):
                g_start(j + 2, b)

        return 0

    lax.fori_loop(0, NCHUNK_S // 2, pair, 0)
    plsc.subcore_barrier()
    for k in range(RPT // CH_S):
        r0 = rbase + k * CH_S
        pltpu.sync_copy(acc.at[pl.ds(r0, CH_S)], rows0)
        pltpu.sync_copy(rows0, out.at[c, pl.ds(r0, CH_S)])


def _make_sc_gather2_impl(sub=False):
    scratch = [
        pltpu.VMEM((NCHUNK, CH), jnp.int32),
        pltpu.VMEM((NCHUNK, CH), jnp.int32),
        pltpu.VMEM((CH, F), f32),
        pltpu.VMEM((CH, F), f32),
        pltpu.VMEM((CH, F), f32),
        pltpu.VMEM((CH, F), f32),
        pltpu.VMEM((CH, F), f32),
        pltpu.VMEM((CH, F), f32),
        pltpu.SemaphoreType.DMA,
        pltpu.SemaphoreType.DMA,
        pltpu.SemaphoreType.DMA,
        pltpu.SemaphoreType.DMA,
        pltpu.SemaphoreType.DMA,
        pltpu.SemaphoreType.DMA,
    ]

    def body(ta, tb, src3, dst3, out, si2, di2, a0, a1, b0, b1, o0, o1,
             as0, as1, bs0, bs1, os0, os1):
        c, s, wid = _worker()
        ebase = wid * EW
        ab = (a0, a1)
        bb = (b0, b1)
        ob = (o0, o1)
        asem = (as0, as1)
        bsem = (bs0, bs1)
        osem = (os0, os1)

        pltpu.sync_copy(src3.at[wid], si2)
        pltpu.sync_copy(dst3.at[wid], di2)

        def g_start(j, b):
            pltpu.async_copy(ta.at[si2.at[j]], ab[b], asem[b])
            pltpu.async_copy(tb.at[di2.at[j]], bb[b], bsem[b])

        def g_wait(j, b):
            pltpu.make_async_copy(ta.at[si2.at[j]], ab[b], asem[b]).wait()
            pltpu.make_async_copy(tb.at[di2.at[j]], bb[b], bsem[b]).wait()

        def op_to(out_ref, x_ref, y_ref):
            nv = F // 16

            def rbody(r, _):
                for cb in range(nv):
                    sl = pl.ds(cb * 16, 16)
                    if sub:
                        out_ref[r, sl] = x_ref[r, sl] - y_ref[r, sl]
                    else:
                        out_ref[r, sl] = x_ref[r, sl] + y_ref[r, sl]
                return 0

            lax.fori_loop(0, CH, rbody, 0, unroll=8)

        g_start(0, 0)
        g_start(1, 1)

        def pair(g, _):
            for b in range(2):
                j = g * 2 + b

                @pl.when(j >= 2)
                def _(j=j, b=b):
                    pltpu.make_async_copy(
                        ob[b], out.at[pl.ds(ebase + (j - 2) * CH, CH)],
                        osem[b]).wait()

                g_wait(j, b)
                op_to(ob[b], ab[b], bb[b])

                @pl.when(j + 2 < NCHUNK)
                def _(j=j, b=b):
                    g_start(j + 2, b)

                pltpu.async_copy(ob[b], out.at[pl.ds(ebase + j * CH, CH)],
                                 osem[b])
            return 0

        lax.fori_loop(0, NCHUNK // 2, pair, 0)
        for b in range(2):
            j = NCHUNK - 2 + b
            pltpu.make_async_copy(ob[b], out.at[pl.ds(ebase + j * CH, CH)],
                                  osem[b]).wait()

    return pl.kernel(
        body,
        out_type=jax.ShapeDtypeStruct((E_PAD, F), f32),
        mesh=_sc_mesh(),
        scratch_types=scratch,
    )


_make_sc_gather2 = functools.cache(_make_sc_gather2_impl)


def _sc_gather2(*args):
    return _make_sc_gather2(False)(*args)


def _sc_gather_diff(*args):
    return _make_sc_gather2(True)(*args)


@functools.cache
def _sc_ones_segsum_k():
    return pl.kernel(
        _sc_ones_segsum_body,
        out_type=jax.ShapeDtypeStruct((NC, N_PAD, F), f32),
        mesh=_sc_mesh(),
        scratch_types=[
            pltpu.VMEM_SHARED((N_PAD, F), f32),
            pltpu.VMEM((NCHUNK_S, CH_S), jnp.int32),
            pltpu.VMEM((CH_S, F), f32),
        ],
    )


def _sc_ones_segsum(*args):
    return _sc_ones_segsum_k()(*args)


def _sc_ones_segsum_body(dst3, out, acc, di2, rows_v):
    c, s, wid = _worker()
    rbase = s * RPT

    _fill(rows_v, 0.0, CH_S)
    for k in range(RPT // CH_S):
        pltpu.sync_copy(rows_v, acc.at[pl.ds(rbase + k * CH_S, CH_S)])
    pltpu.sync_copy(dst3.at[wid], di2)
    plsc.subcore_barrier()
    _fill(rows_v, 1.0, CH_S)

    def chunk(j, _):
        pltpu.sync_copy(rows_v, acc.at[di2.at[j]], add=True)
        return 0

    lax.fori_loop(0, NCHUNK_S, chunk, 0)
    plsc.subcore_barrier()
    for k in range(RPT // CH_S):
        r0 = rbase + k * CH_S
        pltpu.sync_copy(acc.at[pl.ds(r0, CH_S)], rows_v)
        pltpu.sync_copy(rows_v, out.at[c, pl.ds(r0, CH_S)])


B_N = 1024
B_E = 1024
GN = N_PAD // B_N
GE = E_PAD // B_E


def _row_spec(b, w):
    return pl.BlockSpec((b, w), lambda i: (i, 0))


def _full_spec(shape):
    nd = len(shape)
    return pl.BlockSpec(shape, lambda i: (0,) * nd)


def _tc_call(body, full_shapes, out_widths, grid, b_rows, in_widths,
             interpret=False):
    in_specs = [_row_spec(b_rows, w) for w in in_widths]
    in_specs += [_full_spec(s) for s in full_shapes]
    out_specs = [_row_spec(b_rows, w) for w in out_widths]
    return pl.pallas_call(
        body,
        grid=(grid,),
        in_specs=in_specs,
        out_specs=out_specs[0] if len(out_widths) == 1 else tuple(out_specs),
        out_shape=(
            jax.ShapeDtypeStruct((grid * b_rows, out_widths[0]), f32)
            if len(out_widths) == 1
            else tuple(jax.ShapeDtypeStruct((grid * b_rows, w), f32)
                       for w in out_widths)
        ),
        interpret=interpret,
    )


def _dot(a, b):
    return jnp.dot(a, b, preferred_element_type=f32)


def _sigmoid(v):
    return jax.nn.sigmoid(v)


def _silu(v):
    return v * jax.nn.sigmoid(v)


def _ssp_tc(v):
    return (jnp.log(1.0 + jnp.exp(-jnp.abs(v))) + jnp.maximum(v, 0.0)
            - np.float32(np.log(2.0)))


def _t1_body(x_ref, wnp, bnp, wm, bm, h_ref, lin_ref):
    h = _dot(x_ref[...], wnp[...]) + bnp[...]
    h_ref[...] = h
    lin_ref[...] = _dot(h, wm[...]) + bm[...]


def _prep_node(x, wnp, bnp, wm, bm, interpret=False):
    fn = _tc_call(_t1_body, [wnp.shape, bnp.shape, wm.shape, bm.shape],
                  [F, F], GN, B_N, [x.shape[1]], interpret)
    return fn(x, wnp, bnp, wm, bm)


def _t2_body(attr_ref, diff_ref,
             wg1, bg1, wg2, bg2,
             wpf0, bpf0, wrf0, wf20, bf20,
             wpf1, bpf1, wrf1, wf21, bf21,
             eg_ref, f0_ref, f1_ref, d_ref):
    a = attr_ref[...]
    diff = diff_ref[...]
    d2 = jnp.sum(diff * diff, axis=1, keepdims=True)
    d = jnp.sqrt(d2 + 1e-12)
    centers = (4.0 / 63.0) * lax.broadcasted_iota(jnp.int32, (1, NRBF), 1).astype(f32)
    rbf = jnp.exp(-10.0 * (d - centers) ** 2)

    eg1 = jax.nn.relu(_dot(a, wg1[...]) + bg1[...])
    eg_ref[...] = _dot(eg1, wg2[...]) + bg2[...]

    ef0 = _dot(a, wpf0[...]) + _dot(rbf, wrf0[...]) + bpf0[...]
    f0_ref[...] = _dot(_ssp_tc(ef0), wf20[...]) + bf20[...]
    ef1 = _dot(a, wpf1[...]) + _dot(rbf, wrf1[...]) + bpf1[...]
    f1_ref[...] = _dot(_ssp_tc(ef1), wf21[...]) + bf21[...]

    d_ref[...] = jnp.concatenate(
        [d, d2, jnp.zeros((d.shape[0], 14), f32)], axis=1)


def _prep_edge(attr, diff, weights, interpret=False):
    full_shapes = [w.shape for w in weights]
    fn = _tc_call(_t2_body, full_shapes, [F, F, F, 16], GE, B_E,
                  [attr.shape[1], F], interpret)
    return fn(attr, diff, *weights)


def _t3_body(h_ref, m0_ref, m1_ref, dg0_ref, dg1_ref,
             wgi, bgi, wgh, bgh, wnx, bnx, h_out, lin_out):
    h = h_ref[...]
    deg = jnp.maximum(dg0_ref[...][:, :1] + dg1_ref[...][:, :1], 1.0)
    m2d = (m0_ref[...] + m1_ref[...]) / deg
    gi = _dot(m2d, wgi[...]) + bgi[...]
    gh = _dot(h, wgh[...]) + bgh[...]
    r = _sigmoid(gi[:, :F] + gh[:, :F])
    z = _sigmoid(gi[:, F:2 * F] + gh[:, F:2 * F])
    a = jnp.tanh(gi[:, 2 * F:] + r * gh[:, 2 * F:])
    hn = (1.0 - z) * a + z * h
    h_out[...] = hn
    lin_out[...] = _dot(hn, wnx[...]) + bnx[...]


def _gru_step(h, m0, m1, dg0, dg1, wgi, bgi, wgh, bgh, wnx, bnx, interpret=False):
    fn = _tc_call(_t3_body,
                  [wgi.shape, bgi.shape, wgh.shape, bgh.shape, wnx.shape, bnx.shape],
                  [F, F], GN, B_N, [F, F, F, 16, 16], interpret)
    return fn(h, m0, m1, dg0, dg1, wgi, bgi, wgh, bgh, wnx, bnx)


def _t4_body(xs_ref, a0_ref, a1_ref, wu1, bu1, wu2, bu2, wnx, bnx, xs_out, lin_out):
    agg = a0_ref[...] + a1_ref[...]
    t = _ssp_tc(_dot(agg, wu1[...]) + bu1[...])
    xs = xs_ref[...] + _dot(t, wu2[...]) + bu2[...]
    xs_out[...] = xs
    lin_out[...] = _dot(xs, wnx[...]) + bnx[...]


def _schnet_node(xs, a0, a1, wu1, bu1, wu2, bu2, wnx, bnx, interpret=False):
    fn = _tc_call(_t4_body,
                  [wu1.shape, bu1.shape, wu2.shape, bu2.shape, wnx.shape, bnx.shape],
                  [F, F], GN, B_N, [F, F, F], interpret)
    return fn(xs, a0, a1, wu1, bu1, wu2, bu2, wnx, bnx)


def _t5_body(xe_ref, a0_ref, a1_ref, wh1a, wh1b, bh1, wh2, bh2, xe_out):
    xe = xe_ref[...]
    agg = a0_ref[...] + a1_ref[...]
    u = _silu(_dot(xe, wh1a[...]) + _dot(agg, wh1b[...]) + bh1[...])
    xe_out[...] = xe + _dot(u, wh2[...]) + bh2[...]


def _egnn_node(xe, a0, a1, wh1a, wh1b, bh1, wh2, bh2, interpret=False):
    fn = _tc_call(_t5_body,
                  [wh1a.shape, wh1b.shape, bh1.shape, wh2.shape, bh2.shape],
                  [F], GN, B_N, [F, F, F], interpret)
    return fn(xe, a0, a1, wh1a, wh1b, bh1, wh2, bh2)


def _tnm2_body(x_ref, wa, wb, a_out, b_out):
    x = x_ref[...]
    a_out[...] = _dot(x, wa[...])
    b_out[...] = _dot(x, wb[...])


def _node_ab(x, wa, wb, interpret=False):
    fn = _tc_call(_tnm2_body, [wa.shape, wb.shape], [F, F], GN, B_N, [F],
                  interpret)
    return fn(x, wa, wb)


def _t6_body(g_ref, attr_ref, d_ref, wpe, bpe, wre, wd2r, we2, be2, m_ref):
    d = d_ref[...][:, :1]
    d2 = d_ref[...][:, 1:2]
    centers = (4.0 / 63.0) * lax.broadcasted_iota(jnp.int32, (1, NRBF), 1).astype(f32)
    rbf = jnp.exp(-10.0 * (d - centers) ** 2)
    a = attr_ref[...]
    efe = _dot(a, wpe[...]) + _dot(rbf, wre[...]) + bpe[...] + d2 * wd2r[...]
    u = _silu(g_ref[...] + efe)
    m_ref[...] = _silu(_dot(u, we2[...]) + be2[...])


def _egnn_msg(g, attr, d16, wpe, bpe, wre, wd2r, we2, be2, interpret=False):
    fn = _tc_call(_t6_body,
                  [wpe.shape, bpe.shape, wre.shape, wd2r.shape, we2.shape,
                   be2.shape],
                  [F], GE, B_E, [F, attr.shape[1], 16], interpret)
    return fn(g, attr, d16, wpe, bpe, wre, wd2r, we2, be2)


def _t7_body(h_ref, xs_ref, xe_ref, wg3a, wg3b, bg3, wg23a, wg23b, bg23,
             wda, wdb, bd1, p_out, q_out):
    h = h_ref[...]
    xs = xs_ref[...]
    xe = xe_ref[...]
    g3 = _sigmoid(_dot(xs, wg3a[...]) + _dot(xe, wg3b[...]) + bg3[...])
    x3d = g3 * xs + (1.0 - g3) * xe
    g23 = _sigmoid(_dot(h, wg23a[...]) + _dot(x3d, wg23b[...]) + bg23[...])
    x3d = g23 * x3d + (1.0 - g23) * h
    p_out[...] = _dot(x3d, wda[...])
    q_out[...] = _dot(x3d, wdb[...]) + bd1[...]


def _fuse_gates(h, xs, xe, wg3a, wg3b, bg3, wg23a, wg23b, bg23, wda, wdb, bd1,
                interpret=False):
    fn = _tc_call(_t7_body,
                  [wg3a.shape, wg3b.shape, bg3.shape, wg23a.shape, wg23b.shape,
                   bg23.shape, wda.shape, wdb.shape, bd1.shape],
                  [F, F], GN, B_N, [F, F, F], interpret)
    return fn(h, xs, xe, wg3a, wg3b, bg3, wg23a, wg23b, bg23, wda, wdb, bd1)


def _t8_body(pair_ref, d_ref, wd2, bd2, out_ref):
    i = pl.program_id(0)
    v = jax.nn.relu(pair_ref[...])
    pred = jnp.sum(v * wd2[...], axis=1, keepdims=True) + bd2[...]
    err = pred - d_ref[...][:, :1]
    gidx = i * B_E + lax.broadcasted_iota(jnp.int32, (B_E, 1), 0)
    msk = (gidx < E).astype(f32)
    part = jnp.sum(err * err * msk)

    @pl.when(i == 0)
    def _():
        out_ref[0, 0] = 0.0

    out_ref[0, 0] += part


def _loss_sum(pair, d16, wd2, bd2, interpret=False):
    fn = pl.pallas_call(
        _t8_body,
        grid=(GE,),
        in_specs=[
            _row_spec(B_E, F),
            _row_spec(B_E, 16),
            _full_spec(wd2.shape),
            _full_spec(bd2.shape),
        ],
        out_specs=pl.BlockSpec((1, 1), lambda i: (0, 0), memory_space=pltpu.SMEM),
        out_shape=jax.ShapeDtypeStruct((1, 1), f32),
        interpret=interpret,
    )
    return fn(pair, d16, wd2, bd2)


def kernel(x, edge_index, edge_attr, pos, params):
    p = params
    r1 = lambda b: b.reshape(1, -1)

    xp = jnp.pad(x, ((0, N_PAD - N), (0, 0)))
    attr_p = jnp.pad(edge_attr, ((0, E_PAD - E), (0, 0)))
    src_p = jnp.pad(edge_index[0].astype(jnp.int32), (0, E_PAD - E),
                    constant_values=N_PAD - 1)
    dst_p = jnp.pad(edge_index[1].astype(jnp.int32), (0, E_PAD - E),
                    constant_values=N_PAD - 1)
    pos128 = jnp.pad(pos, ((0, N_PAD - N), (0, F - 3)))
    perm = jnp.argsort(src_p)
    src_p = src_p[perm]
    dst_p = dst_p[perm]
    attr_p = attr_p[perm]
    src3 = src_p.reshape(NW, NCHUNK, CH)
    dst3 = dst_p.reshape(NW, NCHUNK, CH)
    dst3s = dst_p.reshape(NW, NCHUNK_S, CH_S)
    sd3f = jnp.concatenate([src_p.reshape(NW, NCHUNK_F, CH_F),
                            dst_p.reshape(NW, NCHUNK_F, CH_F)], axis=-1)

    Wp, bp = p["edge_proj_3d"]["w"], p["edge_proj_3d"]["b"]
    t2_w = [p["edge_gate1"]["w"], r1(p["edge_gate1"]["b"]),
            p["edge_gate2"]["w"], r1(p["edge_gate2"]["b"])]
    for l in p["schnet"]:
        W1, b1 = l["filt1"]["w"], l["filt1"]["b"]
        t2_w += [Wp @ W1[:F], r1(bp @ W1[:F] + b1), W1[F:],
                 l["filt2"]["w"], r1(l["filt2"]["b"])]
    egnn_w = []
    for l in p["egnn"]:
        W1, b1 = l["e1"]["w"], l["e1"]["b"]
        Wtop, Wbot = W1[2 * F + 1:3 * F + 1], W1[3 * F + 1:]
        egnn_w.append([Wp @ Wtop, r1(bp @ Wtop + b1), Wbot, W1[2 * F:2 * F + 1]])

    h, lin = _prep_node(xp, p["node_proj"]["w"], r1(p["node_proj"]["b"]),
                        p["msg"]["w"], r1(p["msg"]["b"]))

    pdiff = _sc_gather_diff(pos128, pos128, src3, dst3)
    degf = _sc_ones_segsum(dst3s)
    dg0, dg1 = degf[0, :, :16], degf[1, :, :16]

    eg, filt0, filt1, d16 = _prep_edge(attr_p, pdiff, t2_w)
    filts = [filt0, filt1]

    winp0, binp0 = p["schnet"][0]["inp"]["w"], r1(p["schnet"][0]["inp"]["b"])
    for step in range(3):
        mh = _sc_gmul_segsum(lin, eg, sd3f)
        if step < 2:
            wnx, bnx = p["msg"]["w"], r1(p["msg"]["b"])
        else:
            wnx, bnx = winp0, binp0
        h, lin = _gru_step(h, mh[0], mh[1], dg0, dg1,
                           p["gru_i"]["w"], r1(p["gru_i"]["b"]),
                           p["gru_h"]["w"], r1(p["gru_h"]["b"]), wnx, bnx)

    xs = h
    winp1, binp1 = p["schnet"][1]["inp"]["w"], r1(p["schnet"][1]["inp"]["b"])
    for li, l in enumerate(p["schnet"]):
        ah = _sc_gmul_segsum(lin, filts[li], sd3f)
        xs, lin = _schnet_node(xs, ah[0], ah[1],
                               l["upd1"]["w"], r1(l["upd1"]["b"]),
                               l["upd2"]["w"], r1(l["upd2"]["b"]),
                               winp1, binp1)

    xe = h
    W1_0 = p["egnn"][0]["e1"]["w"]
    A, B = _node_ab(h, W1_0[:F], W1_0[F:2 * F])
    for li, l in enumerate(p["egnn"]):
        G = _sc_gather2(A, B, src3, dst3)
        mij = _egnn_msg(G, attr_p, d16, *egnn_w[li],
                        l["e2"]["w"], r1(l["e2"]["b"]))
        agg = _sc_segsum(mij, dst3s)
        Wh1 = l["h1"]["w"]
        xe = _egnn_node(xe, agg[0], agg[1], Wh1[:F], Wh1[F:], r1(l["h1"]["b"]),
                        l["h2"]["w"], r1(l["h2"]["b"]))
        if li == 0:
            W1_1 = p["egnn"][1]["e1"]["w"]
            A, B = _node_ab(xe, W1_1[:F], W1_1[F:2 * F])

    Wg3, Wg23, Wd1 = p["gate3"]["w"], p["gate23"]["w"], p["dist1"]["w"]
    P, Q = _fuse_gates(h, xs, xe, Wg3[:F], Wg3[F:], r1(p["gate3"]["b"]),
                       Wg23[:F], Wg23[F:], r1(p["gate23"]["b"]),
                       Wd1[:F], Wd1[F:], r1(p["dist1"]["b"]))
    pair = _sc_gather2(P, Q, src3, dst3)
    ssum = _loss_sum(pair, d16, r1(p["dist2"]["w"][:, 0]),
                     p["dist2"]["b"].reshape(1, 1))
    return ssum[0, 0] / np.float32(E)
